# Initial kernel scaffold; baseline (speedup 1.0000x reference)
import jax
import jax.numpy as jnp
from jax.experimental import pallas as pl


def kernel(pos_u, pos_w, neg_w, u_table, w_table):
    raise NotImplementedError("write your pallas kernel here")


# speedup vs baseline: 1.8153x; 1.8153x over previous
import functools

import jax
import jax.numpy as jnp
from jax import lax
from jax.experimental import pallas as pl
from jax.experimental.pallas import tpu as pltpu
from jax.experimental.pallas import tpu_sc as plsc

VOCAB = 1000000
DIM = 64
B = 16384
CTX = 20
NEG = 5
W = NEG + 1

L = 16
NC, NS = 2, 16
NW = NC * NS
BPW = B // NW
G = 64
NCH = BPW // G
UROWS = G * CTX
WROWS = G * W
UIB = UROWS // 128
WIB = WROWS // 128
ND = DIM // L

_mesh = plsc.VectorSubcoreMesh(core_axis_name="c", subcore_axis_name="s")


@functools.partial(
    pl.kernel,
    out_type=(
        jax.ShapeDtypeStruct((B, DIM), jnp.float32),
        jax.ShapeDtypeStruct((B, DIM), jnp.float32),
        jax.ShapeDtypeStruct((B, DIM), jnp.float32),
    ),
    mesh=_mesh,
    compiler_params=pltpu.CompilerParams(use_tc_tiling_on_sc=False),
    scratch_types=[
        pltpu.VMEM((UIB, 128), jnp.int32),
        pltpu.VMEM((WIB, 128), jnp.int32),
        pltpu.VMEM((UROWS, DIM), jnp.float32),
        pltpu.VMEM((WROWS, DIM), jnp.float32),
        pltpu.VMEM((G, DIM), jnp.float32),
        pltpu.VMEM((G, DIM), jnp.float32),
        pltpu.VMEM((G, DIM), jnp.float32),
        pltpu.SemaphoreType.DMA,
    ],
)
def _sc_gather_pool(uidx_hbm, widx_hbm, u_tab, w_tab, su_hbm, sw_hbm, sn_hbm,
                    uidx_v, widx_v, urows_v, wrows_v, su_v, sw_v, sn_v, gsem):
    wid = lax.axis_index("s") * NC + lax.axis_index("c")
    ebase = wid * BPW

    def chunk(g, carry):
        e0 = ebase + g * G
        q = wid * NCH + g
        pltpu.sync_copy(uidx_hbm.at[q], uidx_v)
        pltpu.sync_copy(widx_hbm.at[q], widx_v)
        for j in range(UIB):
            pltpu.async_copy(u_tab.at[uidx_v.at[j]],
                             urows_v.at[pl.ds(j * 128, 128)], gsem)
        for j in range(WIB):
            pltpu.async_copy(w_tab.at[widx_v.at[j]],
                             wrows_v.at[pl.ds(j * 128, 128)], gsem)
        for j in range(UIB):
            pltpu.make_async_copy(u_tab.at[uidx_v.at[j]],
                                  urows_v.at[pl.ds(j * 128, 128)], gsem).wait()
        for j in range(WIB):
            pltpu.make_async_copy(w_tab.at[widx_v.at[j]],
                                  wrows_v.at[pl.ds(j * 128, 128)], gsem).wait()

        def per_ex(e, carry2):
            zero = jnp.zeros((L,), jnp.float32)

            def uacc(c, accs):
                r = e * CTX + c
                return tuple(accs[d] + urows_v[r, pl.ds(d * L, L)]
                             for d in range(ND))

            su = lax.fori_loop(0, CTX, uacc, (zero,) * ND)

            def nacc(n, accs):
                r = e * W + 1 + n
                return tuple(accs[d] + wrows_v[r, pl.ds(d * L, L)]
                             for d in range(ND))

            sn = lax.fori_loop(0, NEG, nacc, (zero,) * ND)

            for d in range(ND):
                su_v[e, pl.ds(d * L, L)] = su[d]
                sw_v[e, pl.ds(d * L, L)] = wrows_v[e * W, pl.ds(d * L, L)]
                sn_v[e, pl.ds(d * L, L)] = sn[d]
            return carry2

        lax.fori_loop(0, G, per_ex, 0)

        pltpu.sync_copy(su_v, su_hbm.at[pl.ds(e0, G)])
        pltpu.sync_copy(sw_v, sw_hbm.at[pl.ds(e0, G)])
        pltpu.sync_copy(sn_v, sn_hbm.at[pl.ds(e0, G)])
        return carry

    lax.fori_loop(0, NCH, chunk, 0)


TC_BLK = 2048


def _logsig(x):
    return jnp.minimum(x, 0.0) - jnp.log1p(jnp.exp(-jnp.abs(x)))


def _tc_score_body(su_ref, sw_ref, sn_ref, out_ref):
    i = pl.program_id(0)
    su = su_ref[...]
    s2 = jnp.sum(su * sw_ref[...], axis=1)
    n2 = jnp.sum(su * sn_ref[...], axis=1)
    part = jnp.sum(_logsig(s2)) + jnp.sum(_logsig(-n2))

    @pl.when(i == 0)
    def _init():
        out_ref[...] = jnp.zeros_like(out_ref)

    out_ref[...] += part


def kernel(pos_u, pos_w, neg_w, u_table, w_table):
    uidx = pos_u.astype(jnp.int32).reshape(B // G, UIB, 128)
    widx = jnp.concatenate(
        [pos_w.astype(jnp.int32)[:, None], neg_w.astype(jnp.int32)], axis=1
    ).reshape(B // G, WIB, 128)

    su, sw, sn = _sc_gather_pool(uidx, widx, u_table, w_table)

    loss = pl.pallas_call(
        _tc_score_body,
        grid=(B // TC_BLK,),
        in_specs=[pl.BlockSpec((TC_BLK, DIM), lambda i: (i, 0))] * 3,
        out_specs=pl.BlockSpec((1, 1), lambda i: (0, 0)),
        out_shape=jax.ShapeDtypeStruct((1, 1), jnp.float32),
    )(su, sw, sn)
    return -loss[0, 0]
